# Initial kernel scaffold; baseline (speedup 1.0000x reference)
import jax
import jax.numpy as jnp
from jax.experimental import pallas as pl


def kernel(images, feature, W1, b1, Wc, bc, Wb, bb):
    raise NotImplementedError("write your pallas kernel here")


# speedup vs baseline: 12.0568x; 12.0568x over previous
import jax
import jax.numpy as jnp
import numpy as np
from jax.experimental import pallas as pl
from jax.experimental.pallas import tpu as pltpu

B = 2
C = 256
A = 9
H = W = 50
IMG = 800
PRE_NMS = 1000
POST_NMS = 1000
NMS_THRESH = 0.7
MIN_SIZE = 0.001

NT = 2560


def _score_body(t_ref, wc_ref, bias_ref, out_ref):
    t = t_ref[0].astype(jnp.bfloat16)
    out = jnp.dot(t, wc_ref[...], preferred_element_type=jnp.float32)
    out_ref[0] = out + bias_ref[0, 0:1, :48]


def _conv_stage(feature, W1, b1, Wc, bc, Wb, bb):
    t = jax.nn.relu(
        jax.lax.conv_general_dilated(
            feature, W1, (1, 1), "SAME",
            dimension_numbers=("NCHW", "OIHW", "NCHW"))
        + b1[None, :, None, None])
    t2 = jnp.transpose(t, (0, 2, 3, 1)).reshape(B, H * W, C)
    t2 = jnp.pad(t2, ((0, 0), (0, NT - H * W), (0, 0)))

    wc = jnp.concatenate([Wc.reshape(A, C).T, Wb.reshape(4 * A, C).T], axis=1)
    wc = jnp.pad(wc, ((0, 0), (0, 48 - 45))).astype(jnp.bfloat16)
    bias = jnp.zeros((1, 1, 256), jnp.float32)
    bias = bias.at[0, 0, :45].set(jnp.concatenate([bc, bb]))

    y = pl.pallas_call(
        _score_body,
        grid=(B,),
        in_specs=[
            pl.BlockSpec((1, NT, C), lambda b: (b, 0, 0)),
            pl.BlockSpec((C, 48), lambda b: (0, 0)),
            pl.BlockSpec((1, 1, 256), lambda b: (0, 0, 0)),
        ],
        out_specs=pl.BlockSpec((1, NT, 48), lambda b: (b, 0, 0)),
        out_shape=jax.ShapeDtypeStruct((B, NT, 48), jnp.float32),
    )(t2, wc, bias)

    y = y[:, :H * W, :45]
    obj = y[..., :A].reshape(B, H * W * A)
    deltas = y[..., A:45].reshape(B, H * W * A, 4)
    return obj, deltas


K = 1024


def _nms_body(sc_ref, dx_ref, dy_ref, dw_ref, dh_ref, wa_ref, ha_ref,
              cx_ref, cy_ref, out_ref):
    scores = sc_ref[0]
    dx, dy = dx_ref[0], dy_ref[0]
    dw, dh = dw_ref[0], dh_ref[0]
    widths, heights = wa_ref[0], ha_ref[0]
    ctr_x, ctr_y = cx_ref[0], cy_ref[0]

    bbox_clip = float(np.log(1000.0 / 16.0))
    dw = jnp.minimum(dw, bbox_clip)
    dh = jnp.minimum(dh, bbox_clip)
    pcx = dx * widths + ctr_x
    pcy = dy * heights + ctr_y
    pw = jnp.exp(dw) * widths
    ph = jnp.exp(dh) * heights
    x1 = jnp.clip(pcx - 0.5 * pw, 0.0, float(IMG))
    y1 = jnp.clip(pcy - 0.5 * ph, 0.0, float(IMG))
    x2 = jnp.clip(pcx + 0.5 * pw, 0.0, float(IMG))
    y2 = jnp.clip(pcy + 0.5 * ph, 0.0, float(IMG))
    valid = ((x2 - x1) >= MIN_SIZE) & ((y2 - y1) >= MIN_SIZE)
    scores = jnp.where(valid, scores, -jnp.inf)

    def col(v):
        return jax.lax.broadcast_in_dim(v.reshape(K), (K, K), (0,))

    def row(v):
        return jax.lax.broadcast_in_dim(v.reshape(K), (K, K), (1,))

    area = (x2 - x1) * (y2 - y1)
    ltx = jnp.maximum(col(x1), row(x1))
    lty = jnp.maximum(col(y1), row(y1))
    rbx = jnp.minimum(col(x2), row(x2))
    rby = jnp.minimum(col(y2), row(y2))
    inter = jnp.maximum(rbx - ltx, 0.0) * jnp.maximum(rby - lty, 0.0)
    iou = inter / (col(area) + row(area) - inter + 1e-9)

    ii = jax.lax.broadcasted_iota(jnp.int32, (K, K), 0)
    jj = jax.lax.broadcasted_iota(jnp.int32, (K, K), 1)
    sup = ((iou > NMS_THRESH) & (ii < jj)).astype(jnp.bfloat16)

    def step(keep):
        hit = jnp.dot(keep.astype(jnp.bfloat16), sup,
                      preferred_element_type=jnp.float32)
        return (hit <= 0.5).astype(jnp.float32)

    def cond(carry):
        keep, prev_changed = carry
        return prev_changed > 0

    def body(carry):
        keep, _ = carry
        nxt = step(keep)
        return nxt, jnp.sum(jnp.abs(nxt - keep))

    keep0 = jnp.ones((1, K), jnp.float32)
    keep, _ = jax.lax.while_loop(cond, body, (keep0, jnp.float32(1.0)))

    kf = (keep > 0.5) & (scores > -jnp.inf)
    kf_b = kf.astype(jnp.bfloat16)
    nk_b = 1.0 - kf_b
    tri = (ii <= jj).astype(jnp.bfloat16)
    csk = jnp.dot(kf_b, tri, preferred_element_type=jnp.float32)
    csn = jnp.dot(nk_b, tri, preferred_element_type=jnp.float32)
    nkept = csk[0, K - 1]
    pos = jnp.where(kf, csk - 1.0, nkept + csn - 1.0)
    pos = pos.astype(jnp.int32)

    permT = (col(pos.astype(jnp.int32)) == jj).astype(jnp.float32)

    def permute(v):
        return jnp.dot(v, permT, preferred_element_type=jnp.float32)

    out_ref[0, 0] = permute(x1)[0]
    out_ref[0, 1] = permute(y1)[0]
    out_ref[0, 2] = permute(x2)[0]
    out_ref[0, 3] = permute(y2)[0]


def _nms_stage(scores, deltas, idx):
    a = idx % A
    hw = idx // A
    wcol = (hw % W).astype(jnp.float32)
    hrow = (hw // W).astype(jnp.float32)
    scales = np.array([128.0, 256.0, 512.0], np.float32)
    ratios = np.array([0.5, 1.0, 2.0], np.float32)
    hr = np.sqrt(ratios)
    ws_t = jnp.asarray((scales[:, None] * hr[None, :]).reshape(-1))
    hs_t = jnp.asarray((scales[:, None] * (1.0 / hr)[None, :]).reshape(-1))
    widths = ws_t[a]
    heights = hs_t[a]
    ctr_x = wcol * float(IMG // W)
    ctr_y = hrow * float(IMG // H)

    ins = [scores[:, None, :], deltas[..., 0][:, None, :],
           deltas[..., 1][:, None, :], deltas[..., 2][:, None, :],
           deltas[..., 3][:, None, :], widths[:, None, :],
           heights[:, None, :], ctr_x[:, None, :], ctr_y[:, None, :]]
    vec_spec = pl.BlockSpec((1, 1, K), lambda b: (b, 0, 0))
    out = pl.pallas_call(
        _nms_body,
        grid=(B,),
        in_specs=[vec_spec] * 9,
        out_specs=pl.BlockSpec((1, 4, K), lambda b: (b, 0, 0)),
        out_shape=jax.ShapeDtypeStruct((B, 4, K), jnp.float32),
    )(*ins)
    return jnp.transpose(out, (0, 2, 1))


def kernel(images, feature, W1, b1, Wc, bc, Wb, bb):
    obj, deltas = _conv_stage(feature, W1, b1, Wc, bc, Wb, bb)
    scores, top_idx = jax.lax.top_k(obj, PRE_NMS)
    tdeltas = jnp.take_along_axis(deltas, top_idx[..., None], axis=1)
    scores = jnp.pad(scores, ((0, 0), (0, K - PRE_NMS)),
                     constant_values=-jnp.inf)
    tdeltas = jnp.pad(tdeltas, ((0, 0), (0, K - PRE_NMS), (0, 0)))
    top_idx = jnp.pad(top_idx, ((0, 0), (0, K - PRE_NMS)))
    boxes = _nms_stage(scores, tdeltas, top_idx)
    return boxes[:, :POST_NMS]
